# baseline (speedup 1.0000x reference)
import jax
import jax.numpy as jnp
from jax import lax
from jax.experimental import pallas as pl
from jax.experimental.pallas import tpu as pltpu
from jax.experimental.pallas import tpu_sc as plsc

_B = 16384
_EMB = 64
_ROW = 128
_NC = 2
_NS = 16
_NW = _NC * _NS
_BPW = _B // _NW
_CHUNK = 128
_NCHUNK = _BPW // _CHUNK
_HALF = _BPW // 2
_NBIAS = 1000448 // 128


def _mf_body(u_id, i_id, user_emb128, user_bias128, item_emb128,
             item_bias128, mean, out, uidx_v, iidx_v, u_rows, i_rows,
             bias_rows, bu_v, bi_v, out_v, mean_v, sem, bsem):
    wid = lax.axis_index("s") * _NC + lax.axis_index("c")
    base = wid * _BPW

    idx_copies = []
    for c in range(_NCHUNK):
        src = pl.ds(base + c * _CHUNK, _CHUNK)
        idx_copies.append(pltpu.async_copy(u_id.at[src], uidx_v.at[c], sem))
        idx_copies.append(pltpu.async_copy(i_id.at[src], iidx_v.at[c], sem))
    idx_copies.append(pltpu.async_copy(mean, mean_v.at[pl.ds(0, 1)], sem))
    for d in idx_copies:
        d.wait()

    for c in range(_NCHUNK):
        for j in range(_CHUNK // 16):
            s = pl.ds(j * 16, 16)
            uidx_v[c + _NCHUNK, s] = lax.shift_right_logical(uidx_v[c, s], 7)
            iidx_v[c + _NCHUNK, s] = lax.shift_right_logical(iidx_v[c, s], 7)
            uidx_v[c + 2 * _NCHUNK, s] = lax.shift_right_logical(
                uidx_v[c, s], 1)
            iidx_v[c + 2 * _NCHUNK, s] = lax.shift_right_logical(
                iidx_v[c, s], 1)

    mv = mean_v[...]
    mean_vec = jnp.full((16,), mv[0], jnp.float32)
    mask127 = jnp.full((16,), 127, jnp.int32)

    def bias_pass(idx_ref, table, dst_ref):
        for c in range(_NCHUNK):
            pltpu.async_copy(table.at[idx_ref.at[c + _NCHUNK]],
                             bias_rows, bsem).wait()
            for j in range(_CHUNK // 16):
                rows = jnp.full((16,), j * 16, jnp.int32) + lax.iota(
                    jnp.int32, 16)
                ids = idx_ref[c, pl.ds(j * 16, 16)]
                lanes = lax.bitwise_and(ids, mask127)
                v = plsc.load_gather(bias_rows, [rows, lanes])
                dst_ref[pl.ds(c * _CHUNK + j * 16, 16)] = v

    bias_pass(uidx_v, user_bias128, bu_v)
    bias_pass(iidx_v, item_bias128, bi_v)

    for h in range(2):
        gathers = []
        for cc in range(_NCHUNK // 2):
            c = h * (_NCHUNK // 2) + cc
            dst = pl.ds(cc * _CHUNK, _CHUNK)
            gathers.append(pltpu.async_copy(
                user_emb128.at[uidx_v.at[c + 2 * _NCHUNK]],
                u_rows.at[dst], sem))
            gathers.append(pltpu.async_copy(
                item_emb128.at[iidx_v.at[c + 2 * _NCHUNK]],
                i_rows.at[dst], sem))
        for d in gathers:
            d.wait()

        def group_body(g, carry):
            rows = g * 16 + lax.iota(jnp.int32, 16)
            half_chunks = (g * 16) // _CHUNK + h * (_NCHUNK // 2)
            chunk = jnp.full((16,), half_chunks, jnp.int32)
            pos = lax.bitwise_and(rows, jnp.full((16,), 127, jnp.int32))
            uids = plsc.load_gather(uidx_v, [chunk, pos])
            iids = plsc.load_gather(iidx_v, [chunk, pos])
            one16 = jnp.full((16,), 1, jnp.int32)
            uoff = lax.shift_left(lax.bitwise_and(uids, one16), 6)
            ioff = lax.shift_left(lax.bitwise_and(iids, one16), 6)
            accs = [jnp.zeros((16,), jnp.float32) for _ in range(4)]
            for d in range(_EMB):
                dv = jnp.full((16,), d, jnp.int32)
                u = plsc.load_gather(u_rows, [rows, dv + uoff])
                it = plsc.load_gather(i_rows, [rows, dv + ioff])
                accs[d % 4] = accs[d % 4] + u * it
            acc = (accs[0] + accs[1]) + (accs[2] + accs[3])
            off = h * _HALF
            bu = plsc.load_gather(bu_v, [rows + off])
            bi = plsc.load_gather(bi_v, [rows + off])
            res = acc + bu + bi + mean_vec
            plsc.store_scatter(out_v, [rows + off], res)
            return carry

        lax.fori_loop(0, _HALF // 16, group_body, 0)

    pltpu.sync_copy(out_v, out.at[pl.ds(base, _BPW)])


@jax.jit
def kernel(u_id, i_id, user_emb, user_bias, item_emb, item_bias, mean):
    mesh = plsc.VectorSubcoreMesh(
        core_axis_name="c", subcore_axis_name="s",
        num_cores=_NC, num_subcores=_NS)
    user_emb128 = user_emb.reshape(user_emb.shape[0] // 2, 128)
    item_emb128 = item_emb.reshape(item_emb.shape[0] // 2, 128)
    nb = _NBIAS * 128
    user_bias128 = jnp.pad(user_bias[:, 0],
                           (0, nb - user_bias.shape[0])).reshape(_NBIAS, 128)
    item_bias128 = jnp.pad(item_bias[:, 0],
                           (0, nb - item_bias.shape[0])).reshape(_NBIAS, 128)
    f = pl.kernel(
        _mf_body,
        out_type=jax.ShapeDtypeStruct((_B,), jnp.float32),
        mesh=mesh,
        compiler_params=pltpu.CompilerParams(needs_layout_passes=False),
        scratch_types=[
            pltpu.VMEM((3 * _NCHUNK, _CHUNK), jnp.int32),
            pltpu.VMEM((3 * _NCHUNK, _CHUNK), jnp.int32),
            pltpu.VMEM((_HALF, _ROW), jnp.float32),
            pltpu.VMEM((_HALF, _ROW), jnp.float32),
            pltpu.VMEM((_CHUNK, _ROW), jnp.float32),
            pltpu.VMEM((_BPW,), jnp.float32),
            pltpu.VMEM((_BPW,), jnp.float32),
            pltpu.VMEM((_BPW,), jnp.float32),
            pltpu.VMEM((16,), jnp.float32),
            pltpu.SemaphoreType.DMA,
            pltpu.SemaphoreType.DMA,
        ],
    )
    return f(u_id, i_id, user_emb128, user_bias128, item_emb128,
             item_bias128, mean)


# speedup vs baseline: 1.0631x; 1.0631x over previous
import jax
import jax.numpy as jnp
from jax import lax
from jax.experimental import pallas as pl
from jax.experimental.pallas import tpu as pltpu
from jax.experimental.pallas import tpu_sc as plsc

_B = 16384
_EMB = 64
_ROW = 128
_NC = 2
_NS = 16
_NW = _NC * _NS
_BPW = _B // _NW
_CHUNK = 128
_NCHUNK = _BPW // _CHUNK
_HALF = _BPW // 2
_NBIAS = 1000448 // 128


def _mf_body(u_id, i_id, user_emb128, user_bias128, item_emb128,
             item_bias128, mean, out, uidx_v, iidx_v, u_rows, i_rows,
             bias_rows, bu_v, bi_v, out_v, mean_v, sem, bsem):
    wid = lax.axis_index("s") * _NC + lax.axis_index("c")
    base = wid * _BPW

    idx_copies = []
    for c in range(_NCHUNK):
        src = pl.ds(base + c * _CHUNK, _CHUNK)
        idx_copies.append(pltpu.async_copy(u_id.at[src], uidx_v.at[c], sem))
        idx_copies.append(pltpu.async_copy(i_id.at[src], iidx_v.at[c], sem))
    idx_copies.append(pltpu.async_copy(mean, mean_v.at[pl.ds(0, 1)], sem))
    for d in idx_copies:
        d.wait()

    for c in range(_NCHUNK):
        for j in range(_CHUNK // 16):
            s = pl.ds(j * 16, 16)
            uidx_v[c + _NCHUNK, s] = lax.shift_right_logical(uidx_v[c, s], 7)
            iidx_v[c + _NCHUNK, s] = lax.shift_right_logical(iidx_v[c, s], 7)

    mv = mean_v[...]
    mean_vec = jnp.full((16,), mv[0], jnp.float32)
    mask127 = jnp.full((16,), 127, jnp.int32)

    def bias_pass(idx_ref, table, dst_ref):
        for c in range(_NCHUNK):
            pltpu.async_copy(table.at[idx_ref.at[c + _NCHUNK]],
                             bias_rows, bsem).wait()
            for j in range(_CHUNK // 16):
                rows = jnp.full((16,), j * 16, jnp.int32) + lax.iota(
                    jnp.int32, 16)
                ids = idx_ref[c, pl.ds(j * 16, 16)]
                lanes = lax.bitwise_and(ids, mask127)
                v = plsc.load_gather(bias_rows, [rows, lanes])
                dst_ref[pl.ds(c * _CHUNK + j * 16, 16)] = v

    bias_pass(uidx_v, user_bias128, bu_v)
    bias_pass(iidx_v, item_bias128, bi_v)

    for h in range(2):
        gathers = []
        for cc in range(_NCHUNK // 2):
            c = h * (_NCHUNK // 2) + cc
            dst = pl.ds(cc * _CHUNK, _CHUNK)
            gathers.append(pltpu.async_copy(
                user_emb128.at[uidx_v.at[c]], u_rows.at[dst], sem))
            gathers.append(pltpu.async_copy(
                item_emb128.at[iidx_v.at[c]], i_rows.at[dst], sem))
        for d in gathers:
            d.wait()

        def group_body(g, carry):
            rows = g * 16 + lax.iota(jnp.int32, 16)
            accs = [jnp.zeros((16,), jnp.float32) for _ in range(4)]
            for d in range(_EMB):
                dv = jnp.full((16,), d, jnp.int32)
                u = plsc.load_gather(u_rows, [rows, dv])
                it = plsc.load_gather(i_rows, [rows, dv])
                accs[d % 4] = accs[d % 4] + u * it
            acc = (accs[0] + accs[1]) + (accs[2] + accs[3])
            off = h * _HALF
            bu = plsc.load_gather(bu_v, [rows + off])
            bi = plsc.load_gather(bi_v, [rows + off])
            res = acc + bu + bi + mean_vec
            plsc.store_scatter(out_v, [rows + off], res)
            return carry

        lax.fori_loop(0, _HALF // 16, group_body, 0)

    pltpu.sync_copy(out_v, out.at[pl.ds(base, _BPW)])


@jax.jit
def kernel(u_id, i_id, user_emb, user_bias, item_emb, item_bias, mean):
    mesh = plsc.VectorSubcoreMesh(
        core_axis_name="c", subcore_axis_name="s",
        num_cores=_NC, num_subcores=_NS)
    user_emb128 = jnp.pad(user_emb, ((0, 0), (0, _ROW - _EMB)))
    item_emb128 = jnp.pad(item_emb, ((0, 0), (0, _ROW - _EMB)))
    nb = _NBIAS * 128
    user_bias128 = jnp.pad(user_bias[:, 0],
                           (0, nb - user_bias.shape[0])).reshape(_NBIAS, 128)
    item_bias128 = jnp.pad(item_bias[:, 0],
                           (0, nb - item_bias.shape[0])).reshape(_NBIAS, 128)
    f = pl.kernel(
        _mf_body,
        out_type=jax.ShapeDtypeStruct((_B,), jnp.float32),
        mesh=mesh,
        compiler_params=pltpu.CompilerParams(needs_layout_passes=False),
        scratch_types=[
            pltpu.VMEM((2 * _NCHUNK, _CHUNK), jnp.int32),
            pltpu.VMEM((2 * _NCHUNK, _CHUNK), jnp.int32),
            pltpu.VMEM((_HALF, _ROW), jnp.float32),
            pltpu.VMEM((_HALF, _ROW), jnp.float32),
            pltpu.VMEM((_CHUNK, _ROW), jnp.float32),
            pltpu.VMEM((_BPW,), jnp.float32),
            pltpu.VMEM((_BPW,), jnp.float32),
            pltpu.VMEM((_BPW,), jnp.float32),
            pltpu.VMEM((16,), jnp.float32),
            pltpu.SemaphoreType.DMA,
            pltpu.SemaphoreType.DMA,
        ],
    )
    return f(u_id, i_id, user_emb128, user_bias128, item_emb128,
             item_bias128, mean)


# speedup vs baseline: 4.1442x; 3.8983x over previous
import jax
import jax.numpy as jnp
from jax import lax
from jax.experimental import pallas as pl
from jax.experimental.pallas import tpu as pltpu
from jax.experimental.pallas import tpu_sc as plsc

_B = 16384
_EMB = 64
_NC = 2
_NS = 16
_NW = _NC * _NS
_BPW = _B // _NW
_CHUNK = 128
_NCHUNK = _BPW // _CHUNK
_NROWS = 1000000
_NSLAB = (_NROWS + 127) // 128
_TAIL = _NSLAB - 1
_TAILBASE = _TAIL * 128
_SPB = (_NSLAB + _NW - 1) // _NW
_NBIAS = 1000448 // 128


def _i16(v):
    return jnp.full((16,), v, jnp.int32)


def _iota16():
    return lax.iota(jnp.int32, 16)


def _sget(ref, i):
    return plsc.load_gather(ref, [_i16(0) + i])[0]


def _extract_body(ids_hbm, tab, tail_in, col_out, ids_v, memb_pos, memb_id,
                  smemb_pos, smemb_id, counts_v, offs_v, slist, sstart,
                  send, slab_buf, tail_buf, colrow, sem, sem2, mcount0):
    wid = lax.axis_index("s") * _NC + lax.axis_index("c")
    lo = wid * _SPB
    hi = jnp.minimum(lo + _SPB, _NSLAB)

    pltpu.async_copy(ids_hbm, ids_v, sem).wait()

    for i in range(17):
        counts_v[pl.ds(i * 16, 16)] = jnp.zeros((16,), jnp.int32)

    def compress(k, cnt):
        ids = ids_v[pl.ds(k * 16, 16)]
        s = lax.shift_right_logical(ids, 7)
        m = jnp.logical_and(s >= lo, s < hi)
        plsc.store_compressed(memb_pos.at[pl.ds(cnt, 16)],
                              k * 16 + _iota16(), mask=m)
        plsc.store_compressed(memb_id.at[pl.ds(cnt, 16)], ids, mask=m)
        return cnt + plsc.all_reduce_population_count(m)[0]

    cnt = lax.fori_loop(0, _B // 16, compress, 0)
    nchunks = lax.shift_right_logical(cnt + 15, 4)
    ones = _i16(1)

    def count_pass(j, carry):
        mids = memb_id[pl.ds(j * 16, 16)]
        valid = (j * 16 + _iota16()) < cnt
        sl = jnp.where(valid, lax.shift_right_logical(mids, 7) - lo, 0)
        for l in range(16):
            ml = jnp.logical_and(valid, _iota16() == l)
            plsc.addupdate_scatter(counts_v, [sl], ones, mask=ml)
        return carry

    lax.fori_loop(0, nchunks, count_pass, 0)

    carry = 0
    for i in range(16):
        c = counts_v[pl.ds(i * 16, 16)]
        inc = plsc.cumsum(c)
        offs_v[pl.ds(i * 16, 16)] = inc - c + carry
        carry = carry + inc[15]
    offs_v[pl.ds(256, 16)] = _i16(0) + carry

    def place_pass(j, carry):
        mids = memb_id[pl.ds(j * 16, 16)]
        mpos = memb_pos[pl.ds(j * 16, 16)]
        valid = (j * 16 + _iota16()) < cnt
        sl = jnp.where(valid, lax.shift_right_logical(mids, 7) - lo, 0)
        for l in range(16):
            ml = jnp.logical_and(valid, _iota16() == l)
            dst = plsc.load_gather(offs_v, [sl])
            plsc.store_scatter(smemb_pos, [dst], mpos, mask=ml)
            plsc.store_scatter(smemb_id, [dst], mids, mask=ml)
            plsc.addupdate_scatter(offs_v, [sl], ones, mask=ml)
        return carry

    lax.fori_loop(0, nchunks, place_pass, 0)

    def compact(oc, nc):
        endv = offs_v[pl.ds(oc * 16, 16)]
        cntv = counts_v[pl.ds(oc * 16, 16)]
        slabs = lo + oc * 16 + _iota16()
        m = jnp.logical_and(cntv > 0, slabs != _TAIL)
        plsc.store_compressed(slist.at[pl.ds(nc, 16)], slabs, mask=m)
        plsc.store_compressed(sstart.at[pl.ds(nc, 16)], endv - cntv, mask=m)
        plsc.store_compressed(send.at[pl.ds(nc, 16)], endv, mask=m)
        return nc + plsc.all_reduce_population_count(m)[0]

    nslabs = lax.fori_loop(0, 16, compact, 0)

    def member_loop(st, en, slot, mcount, from_tail=False):
        def member(k, mc):
            pos = smemb_pos[pl.ds(k, 16)][0]
            idv = smemb_id[pl.ds(k, 16)][0]
            mslot = lax.bitwise_and(mc, 15)
            for q in range(4):
                cv = _iota16() + 16 * q
                if from_tail:
                    p = _i16((idv - _TAILBASE) * _EMB) + cv
                    colq = plsc.load_gather(
                        tail_buf, [lax.shift_right_logical(p, 7),
                                   lax.bitwise_and(p, 127)])
                else:
                    lane = lax.bitwise_and(idv, 127)
                    colq = plsc.load_gather(
                        slab_buf,
                        [_i16(0) + slot, lax.shift_right_logical(cv, 3),
                         lax.bitwise_and(cv, 7), _i16(lane)])
                plsc.store_scatter(colrow, [_i16(0) + mslot, cv], colq)

            @pl.when(mc >= 16)
            def _():
                pltpu.make_async_copy(colrow.at[0],
                                      col_out.at[pl.ds(0, _EMB)],
                                      sem2).wait()

            pltpu.async_copy(colrow.at[mslot],
                             col_out.at[pl.ds(pos * _EMB, _EMB)], sem2)
            return mc + 1

        return lax.fori_loop(st, en, member, mcount)

    tl = _TAIL - lo
    in_range = jnp.logical_and(tl >= 0, tl < _SPB)
    tidx = jnp.where(in_range, tl, 0)
    t_end = _sget(offs_v, tidx)
    t_cnt = _sget(counts_v, tidx)
    t_st = jnp.where(in_range, t_end - t_cnt, 0)
    t_end = jnp.where(in_range, t_end, 0)

    @pl.when(t_end > t_st)
    def _():
        pltpu.sync_copy(tail_in, tail_buf)

    mcount = member_loop(t_st, t_end, 0, mcount0, from_tail=True)

    def fire_slab(t):
        s = _sget(slist, t)
        slot = lax.bitwise_and(t, 3)
        off = pl.multiple_of(s * 128, 128)
        for b0 in range(8):
            pltpu.async_copy(tab.at[pl.ds(8 * b0, 8), pl.ds(off, 128)],
                             slab_buf.at[slot, b0], sem)

    def prologue(i, carry):
        fire_slab(i)
        return carry

    lax.fori_loop(0, jnp.minimum(nslabs, 4), prologue, 0)

    def slab_step(t, mcount):
        for b0 in range(8):
            pltpu.make_async_copy(tab.at[pl.ds(0, 8), pl.ds(0, 128)],
                                  slab_buf.at[0, 0], sem).wait()
        st = _sget(sstart, t)
        en = _sget(send, t)
        mcount = member_loop(st, en, lax.bitwise_and(t, 3), mcount)

        @pl.when(t + 4 < nslabs)
        def _():
            fire_slab(t + 4)

        return mcount

    mcount = lax.fori_loop(0, nslabs, slab_step, mcount)
    return mcount


def _stream_body(u_id, i_id, uT, iT, utail, itail, colU, colI, ids_v,
                 memb_pos, memb_id, smemb_pos, smemb_id, counts_v, offs_v,
                 slist, sstart, send, slab_buf, tail_buf, colrow, sem, sem2):
    mcount = _extract_body(u_id, uT, utail, colU, ids_v, memb_pos, memb_id,
                           smemb_pos, smemb_id, counts_v, offs_v, slist,
                           sstart, send, slab_buf, tail_buf, colrow,
                           sem, sem2, 0)
    mcount = _extract_body(i_id, iT, itail, colI, ids_v, memb_pos, memb_id,
                           smemb_pos, smemb_id, counts_v, offs_v, slist,
                           sstart, send, slab_buf, tail_buf, colrow,
                           sem, sem2, mcount)

    def drain(r, carry):
        pltpu.make_async_copy(colrow.at[0], colU.at[pl.ds(0, _EMB)],
                              sem2).wait()
        return carry

    lax.fori_loop(0, jnp.minimum(mcount, 16), drain, 0)


def _dot_body(u_id, i_id, colU, colI, user_bias128, item_bias128, mean,
              out, uidx_v, iidx_v, u_rows, i_rows, bias_rows, bu_v, bi_v,
              out_v, mean_v, sem, bsem):
    wid = lax.axis_index("s") * _NC + lax.axis_index("c")
    base = wid * _BPW

    cps = [pltpu.async_copy(colU.at[pl.ds(base * _EMB, _BPW * _EMB)],
                            u_rows, sem),
           pltpu.async_copy(colI.at[pl.ds(base * _EMB, _BPW * _EMB)],
                            i_rows, sem),
           pltpu.async_copy(mean, mean_v.at[pl.ds(0, 1)], sem)]
    for c in range(_NCHUNK):
        src = pl.ds(base + c * _CHUNK, _CHUNK)
        cps.append(pltpu.async_copy(u_id.at[src], uidx_v.at[c], sem))
        cps.append(pltpu.async_copy(i_id.at[src], iidx_v.at[c], sem))
    for d in cps:
        d.wait()

    for c in range(_NCHUNK):
        for j in range(_CHUNK // 16):
            s = pl.ds(j * 16, 16)
            uidx_v[c + _NCHUNK, s] = lax.shift_right_logical(uidx_v[c, s], 7)
            iidx_v[c + _NCHUNK, s] = lax.shift_right_logical(iidx_v[c, s], 7)

    mv = mean_v[...]
    mean_vec = jnp.full((16,), mv[0], jnp.float32)
    mask127 = _i16(127)

    def bias_pass(idx_ref, table, dst_ref):
        for c in range(_NCHUNK):
            pltpu.async_copy(table.at[idx_ref.at[c + _NCHUNK]],
                             bias_rows, bsem).wait()
            for j in range(_CHUNK // 16):
                rows = _i16(j * 16) + _iota16()
                ids = idx_ref[c, pl.ds(j * 16, 16)]
                lanes = lax.bitwise_and(ids, mask127)
                v = plsc.load_gather(bias_rows, [rows, lanes])
                dst_ref[pl.ds(c * _CHUNK + j * 16, 16)] = v

    bias_pass(uidx_v, user_bias128, bu_v)
    bias_pass(iidx_v, item_bias128, bi_v)

    def group_body(g, carry):
        rows = g * 16 + _iota16()
        rows64 = rows * _EMB
        accs = [jnp.zeros((16,), jnp.float32) for _ in range(4)]
        for d in range(_EMB):
            u = plsc.load_gather(u_rows, [rows64 + d])
            it = plsc.load_gather(i_rows, [rows64 + d])
            accs[d % 4] = accs[d % 4] + u * it
        acc = (accs[0] + accs[1]) + (accs[2] + accs[3])
        bu = plsc.load_gather(bu_v, [rows])
        bi = plsc.load_gather(bi_v, [rows])
        res = acc + bu + bi + mean_vec
        plsc.store_scatter(out_v, [rows], res)
        return carry

    lax.fori_loop(0, _BPW // 16, group_body, 0)
    pltpu.sync_copy(out_v, out.at[pl.ds(base, _BPW)])


@jax.jit
def kernel(u_id, i_id, user_emb, user_bias, item_emb, item_bias, mean):
    mesh = plsc.VectorSubcoreMesh(
        core_axis_name="c", subcore_axis_name="s",
        num_cores=_NC, num_subcores=_NS)
    cp = pltpu.CompilerParams(needs_layout_passes=False)
    uT = user_emb.T
    iT = item_emb.T
    utail = user_emb[_TAILBASE:].reshape(32, 128)
    itail = item_emb[_TAILBASE:].reshape(32, 128)
    nb = _NBIAS * 128
    user_bias128 = jnp.pad(user_bias[:, 0],
                           (0, nb - user_bias.shape[0])).reshape(_NBIAS, 128)
    item_bias128 = jnp.pad(item_bias[:, 0],
                           (0, nb - item_bias.shape[0])).reshape(_NBIAS, 128)

    stream = pl.kernel(
        _stream_body,
        out_type=(jax.ShapeDtypeStruct((_B * _EMB,), jnp.float32),
                  jax.ShapeDtypeStruct((_B * _EMB,), jnp.float32)),
        mesh=mesh,
        compiler_params=cp,
        scratch_types=[
            pltpu.VMEM((_B,), jnp.int32),
            pltpu.VMEM((_B + 16,), jnp.int32),
            pltpu.VMEM((_B + 16,), jnp.int32),
            pltpu.VMEM((_B + 16,), jnp.int32),
            pltpu.VMEM((_B + 16,), jnp.int32),
            pltpu.VMEM((272,), jnp.int32),
            pltpu.VMEM((272,), jnp.int32),
            pltpu.VMEM((288,), jnp.int32),
            pltpu.VMEM((288,), jnp.int32),
            pltpu.VMEM((288,), jnp.int32),
            pltpu.VMEM((4, 8, 8, 128), jnp.float32),
            pltpu.VMEM((32, 128), jnp.float32),
            pltpu.VMEM((16, _EMB), jnp.float32),
            pltpu.SemaphoreType.DMA,
            pltpu.SemaphoreType.DMA,
        ],
    )
    colU, colI = stream(u_id, i_id, uT, iT, utail, itail)

    dot = pl.kernel(
        _dot_body,
        out_type=jax.ShapeDtypeStruct((_B,), jnp.float32),
        mesh=mesh,
        compiler_params=cp,
        scratch_types=[
            pltpu.VMEM((2 * _NCHUNK, _CHUNK), jnp.int32),
            pltpu.VMEM((2 * _NCHUNK, _CHUNK), jnp.int32),
            pltpu.VMEM((_BPW * _EMB,), jnp.float32),
            pltpu.VMEM((_BPW * _EMB,), jnp.float32),
            pltpu.VMEM((_CHUNK, 128), jnp.float32),
            pltpu.VMEM((_BPW,), jnp.float32),
            pltpu.VMEM((_BPW,), jnp.float32),
            pltpu.VMEM((_BPW,), jnp.float32),
            pltpu.VMEM((16,), jnp.float32),
            pltpu.SemaphoreType.DMA,
            pltpu.SemaphoreType.DMA,
        ],
    )
    return dot(u_id, i_id, colU, colI, user_bias128, item_bias128, mean)
